# baseline (speedup 1.0000x reference)
import jax
import jax.numpy as jnp
from jax.experimental import pallas as pl
from jax.experimental.pallas import tpu as pltpu

_INPUT_DIM = 3072
_HIDDEN_DIM = 4096
_TIME_STEPS = 30
_TAU_MEM = 100.0
_THRESHOLD = 0.5
_INPUT_SCALE = 16.0
_NBLK = 16
_NB = _HIDDEN_DIM // _NBLK


def _body(x_ref, w1_ref, w2_ref, d_ref, mr_ref, m1_ref, m2_ref,
          sra_ref, s1a_ref, a_ref, vr_ref, vc_ref):
    s = pl.program_id(0)
    decay = d_ref[0, 0]
    T = _TIME_STEPS
    B = x_ref.shape[0]
    K1 = x_ref.shape[1]
    K2 = s1a_ref.shape[2]

    @pl.when(s == 0)
    def _retina():
        x = x_ref[...]
        vr_ref[...] = jnp.zeros_like(vr_ref)
        mr_ref[...] = jnp.zeros_like(mr_ref)

        def rstep(t, c):
            vr = vr_ref[...] * decay + x
            sr = (vr > _THRESHOLD).astype(jnp.float32)
            vr_ref[...] = vr * (1.0 - sr)
            mr_ref[...] += sr
            sra_ref[t] = sr.astype(jnp.bfloat16)
            return c

        jax.lax.fori_loop(0, T, rstep, 0)
        mr_ref[...] = mr_ref[...] / jnp.float32(T)

    @pl.when(s < _NBLK)
    def _dot1():
        lhs = sra_ref[...].reshape(T * B, K1)
        w1b = w1_ref[...].astype(jnp.bfloat16)
        a_ref[s % 2] = jnp.dot(
            lhs, w1b, preferred_element_type=jnp.float32).reshape(T, B, _NB)

    @pl.when((s >= 1) & (s <= _NBLK))
    def _recur1():
        col = (s - 1) * _NB
        buf = (s - 1) % 2
        vc_ref[...] = jnp.zeros_like(vc_ref)
        m1_ref[...] = jnp.zeros_like(m1_ref)

        def lstep(t, c):
            v1 = vc_ref[...] * decay + a_ref[buf, t]
            s1 = (v1 > _THRESHOLD).astype(jnp.float32)
            vc_ref[...] = v1 * (1.0 - s1)
            m1_ref[...] += s1
            s1a_ref[t, :, pl.ds(col, _NB)] = s1.astype(jnp.bfloat16)
            return c

        jax.lax.fori_loop(0, T, lstep, 0)
        m1_ref[...] = m1_ref[...] / jnp.float32(T)

    @pl.when((s >= _NBLK) & (s < 2 * _NBLK))
    def _dot2():
        lhs = s1a_ref[...].reshape(T * B, K2)
        w2b = w2_ref[...].astype(jnp.bfloat16)
        a_ref[(s - _NBLK) % 2] = jnp.dot(
            lhs, w2b, preferred_element_type=jnp.float32).reshape(T, B, _NB)

    @pl.when(s >= _NBLK + 1)
    def _recur2():
        buf = (s - _NBLK - 1) % 2
        vc_ref[...] = jnp.zeros_like(vc_ref)
        m2_ref[...] = jnp.zeros_like(m2_ref)

        def lstep(t, c):
            v2 = vc_ref[...] * decay + a_ref[buf, t]
            s2 = (v2 > _THRESHOLD).astype(jnp.float32)
            vc_ref[...] = v2 * (1.0 - s2)
            m2_ref[...] += s2
            return c

        jax.lax.fori_loop(0, T, lstep, 0)
        m2_ref[...] = m2_ref[...] / jnp.float32(T)


def kernel(x, W1, W2):
    B = x.shape[0]
    T = _TIME_STEPS
    f32 = jnp.float32
    bf16 = jnp.bfloat16
    xmax = jnp.max(x)
    xn = jnp.where(xmax > 0, x / xmax, x) * _INPUT_SCALE
    decay = jnp.exp(jnp.float32(-1.0 / _TAU_MEM)).reshape(1, 1)

    nb1 = _NBLK - 1

    mr, m1, m2 = pl.pallas_call(
        _body,
        grid=(2 * _NBLK + 1,),
        in_specs=[
            pl.BlockSpec((B, _INPUT_DIM), lambda s: (0, 0)),
            pl.BlockSpec((_INPUT_DIM, _NB),
                         lambda s: (0, jnp.minimum(s, nb1))),
            pl.BlockSpec((_HIDDEN_DIM, _NB),
                         lambda s: (0, jnp.clip(s - _NBLK, 0, nb1))),
            pl.BlockSpec((1, 1), lambda s: (0, 0)),
        ],
        out_specs=[
            pl.BlockSpec((B, _INPUT_DIM), lambda s: (0, 0)),
            pl.BlockSpec((B, _NB), lambda s: (0, jnp.clip(s - 1, 0, nb1))),
            pl.BlockSpec((B, _NB),
                         lambda s: (0, jnp.clip(s - _NBLK - 1, 0, nb1))),
        ],
        out_shape=[
            jax.ShapeDtypeStruct((B, _INPUT_DIM), f32),
            jax.ShapeDtypeStruct((B, _HIDDEN_DIM), f32),
            jax.ShapeDtypeStruct((B, _HIDDEN_DIM), f32),
        ],
        scratch_shapes=[
            pltpu.VMEM((T, B, _INPUT_DIM), bf16),
            pltpu.VMEM((T, B, _HIDDEN_DIM), bf16),
            pltpu.VMEM((2, T, B, _NB), f32),
            pltpu.VMEM((B, _INPUT_DIM), f32),
            pltpu.VMEM((B, _NB), f32),
        ],
        compiler_params=pltpu.CompilerParams(
            dimension_semantics=("arbitrary",),
            vmem_limit_bytes=64 * 1024 * 1024,
        ),
    )(xn, W1, W2, decay)

    return (mr, m1, m2)


# speedup vs baseline: 1.0434x; 1.0434x over previous
import jax
import jax.numpy as jnp
from jax.experimental import pallas as pl
from jax.experimental.pallas import tpu as pltpu

_INPUT_DIM = 3072
_HIDDEN_DIM = 4096
_TIME_STEPS = 30
_TAU_MEM = 100.0
_THRESHOLD = 0.5
_INPUT_SCALE = 16.0
_NBLK = 16
_NB = _HIDDEN_DIM // _NBLK


def _body(x_ref, w1_ref, w2_ref, d_ref, mr_ref, m1_ref, m2_ref,
          sra_ref, s1a_ref, a_ref, vr_ref, vc_ref):
    s = pl.program_id(0)
    decay = d_ref[0, 0]
    T = _TIME_STEPS
    B = x_ref.shape[0]
    K1 = x_ref.shape[1]
    K2 = s1a_ref.shape[2]

    @pl.when(s == 0)
    def _retina():
        x = x_ref[...]
        vr_ref[...] = jnp.zeros_like(vr_ref)
        mr_ref[...] = jnp.zeros_like(mr_ref)

        def rstep(t, c):
            vr = vr_ref[...] * decay + x
            mask = vr > _THRESHOLD
            sr = jnp.where(mask, 1.0, 0.0).astype(jnp.float32)
            vr_ref[...] = jnp.where(mask, 0.0, vr)
            mr_ref[...] += sr
            sra_ref[t] = sr.astype(jnp.bfloat16)
            return c

        jax.lax.fori_loop(0, T, rstep, 0, unroll=3)
        mr_ref[...] = mr_ref[...] / jnp.float32(T)

    @pl.when(s < _NBLK)
    def _dot1():
        lhs = sra_ref[...].reshape(T * B, K1)
        w1b = w1_ref[...].astype(jnp.bfloat16)
        a_ref[s % 2] = jnp.dot(
            lhs, w1b, preferred_element_type=jnp.float32).reshape(T, B, _NB)

    @pl.when((s >= 1) & (s <= _NBLK))
    def _recur1():
        col = (s - 1) * _NB
        buf = (s - 1) % 2
        vc_ref[...] = jnp.zeros_like(vc_ref)
        m1_ref[...] = jnp.zeros_like(m1_ref)

        def lstep(t, c):
            v1 = vc_ref[...] * decay + a_ref[buf, t]
            mask = v1 > _THRESHOLD
            s1 = jnp.where(mask, 1.0, 0.0).astype(jnp.float32)
            vc_ref[...] = jnp.where(mask, 0.0, v1)
            m1_ref[...] += s1
            s1a_ref[t, :, pl.ds(col, _NB)] = s1.astype(jnp.bfloat16)
            return c

        jax.lax.fori_loop(0, T, lstep, 0, unroll=3)
        m1_ref[...] = m1_ref[...] / jnp.float32(T)

    @pl.when((s >= _NBLK) & (s < 2 * _NBLK))
    def _dot2():
        lhs = s1a_ref[...].reshape(T * B, K2)
        w2b = w2_ref[...].astype(jnp.bfloat16)
        a_ref[(s - _NBLK) % 2] = jnp.dot(
            lhs, w2b, preferred_element_type=jnp.float32).reshape(T, B, _NB)

    @pl.when(s >= _NBLK + 1)
    def _recur2():
        buf = (s - _NBLK - 1) % 2
        vc_ref[...] = jnp.zeros_like(vc_ref)
        m2_ref[...] = jnp.zeros_like(m2_ref)

        def lstep(t, c):
            v2 = vc_ref[...] * decay + a_ref[buf, t]
            mask = v2 > _THRESHOLD
            s2 = jnp.where(mask, 1.0, 0.0).astype(jnp.float32)
            vc_ref[...] = jnp.where(mask, 0.0, v2)
            m2_ref[...] += s2
            return c

        jax.lax.fori_loop(0, T, lstep, 0, unroll=3)
        m2_ref[...] = m2_ref[...] / jnp.float32(T)


def kernel(x, W1, W2):
    B = x.shape[0]
    T = _TIME_STEPS
    f32 = jnp.float32
    bf16 = jnp.bfloat16
    xmax = jnp.max(x)
    xn = jnp.where(xmax > 0, x / xmax, x) * _INPUT_SCALE
    decay = jnp.exp(jnp.float32(-1.0 / _TAU_MEM)).reshape(1, 1)

    nb1 = _NBLK - 1

    mr, m1, m2 = pl.pallas_call(
        _body,
        grid=(2 * _NBLK + 1,),
        in_specs=[
            pl.BlockSpec((B, _INPUT_DIM), lambda s: (0, 0)),
            pl.BlockSpec((_INPUT_DIM, _NB),
                         lambda s: (0, jnp.minimum(s, nb1))),
            pl.BlockSpec((_HIDDEN_DIM, _NB),
                         lambda s: (0, jnp.clip(s - _NBLK, 0, nb1))),
            pl.BlockSpec((1, 1), lambda s: (0, 0)),
        ],
        out_specs=[
            pl.BlockSpec((B, _INPUT_DIM), lambda s: (0, 0)),
            pl.BlockSpec((B, _NB), lambda s: (0, jnp.clip(s - 1, 0, nb1))),
            pl.BlockSpec((B, _NB),
                         lambda s: (0, jnp.clip(s - _NBLK - 1, 0, nb1))),
        ],
        out_shape=[
            jax.ShapeDtypeStruct((B, _INPUT_DIM), f32),
            jax.ShapeDtypeStruct((B, _HIDDEN_DIM), f32),
            jax.ShapeDtypeStruct((B, _HIDDEN_DIM), f32),
        ],
        scratch_shapes=[
            pltpu.VMEM((T, B, _INPUT_DIM), bf16),
            pltpu.VMEM((T, B, _HIDDEN_DIM), bf16),
            pltpu.VMEM((2, T, B, _NB), f32),
            pltpu.VMEM((B, _INPUT_DIM), f32),
            pltpu.VMEM((B, _NB), f32),
        ],
        compiler_params=pltpu.CompilerParams(
            dimension_semantics=("arbitrary",),
            vmem_limit_bytes=64 * 1024 * 1024,
        ),
    )(xn, W1, W2, decay)

    return (mr, m1, m2)


# speedup vs baseline: 1.0808x; 1.0358x over previous
import jax
import jax.numpy as jnp
from jax.experimental import pallas as pl
from jax.experimental.pallas import tpu as pltpu

_INPUT_DIM = 3072
_HIDDEN_DIM = 4096
_TIME_STEPS = 30
_TAU_MEM = 100.0
_THRESHOLD = 0.5
_INPUT_SCALE = 16.0
_NBLK = 16
_NB = _HIDDEN_DIM // _NBLK


def _body(x_ref, w1_ref, w2_ref, d_ref, mr_ref, m1_ref, m2_ref,
          sra_ref, s1a_ref, a_ref, vr_ref, vc_ref):
    s = pl.program_id(0)
    decay = d_ref[0, 0]
    T = _TIME_STEPS
    B = x_ref.shape[0]
    K1 = x_ref.shape[1]
    K2 = s1a_ref.shape[2]

    @pl.when(s == 0)
    def _retina():
        xr = x_ref[...]
        xmax = jnp.max(xr)
        x = jnp.where(xmax > 0, xr / xmax, xr) * _INPUT_SCALE
        vr_ref[...] = jnp.zeros_like(vr_ref)
        mr_ref[...] = jnp.zeros_like(mr_ref)

        def rstep(t, c):
            vr = vr_ref[...] * decay + x
            mask = vr > _THRESHOLD
            sr = jnp.where(mask, 1.0, 0.0).astype(jnp.float32)
            vr_ref[...] = jnp.where(mask, 0.0, vr)
            mr_ref[...] += sr
            sra_ref[t] = sr.astype(jnp.bfloat16)
            return c

        jax.lax.fori_loop(0, T, rstep, 0, unroll=3)
        mr_ref[...] = mr_ref[...] / jnp.float32(T)

    @pl.when(s < _NBLK)
    def _dot1():
        lhs = sra_ref[...].reshape(T * B, K1)
        w1b = w1_ref[...].astype(jnp.bfloat16)
        a_ref[s % 2] = jnp.dot(
            lhs, w1b, preferred_element_type=jnp.float32).reshape(T, B, _NB)

    @pl.when((s >= 1) & (s <= _NBLK))
    def _recur1():
        col = (s - 1) * _NB
        buf = (s - 1) % 2
        vc_ref[...] = jnp.zeros_like(vc_ref)
        m1_ref[...] = jnp.zeros_like(m1_ref)

        def lstep(t, c):
            v1 = vc_ref[...] * decay + a_ref[buf, t]
            mask = v1 > _THRESHOLD
            s1 = jnp.where(mask, 1.0, 0.0).astype(jnp.float32)
            vc_ref[...] = jnp.where(mask, 0.0, v1)
            m1_ref[...] += s1
            s1a_ref[t, :, pl.ds(col, _NB)] = s1.astype(jnp.bfloat16)
            return c

        jax.lax.fori_loop(0, T, lstep, 0, unroll=3)
        m1_ref[...] = m1_ref[...] / jnp.float32(T)

    @pl.when((s >= _NBLK) & (s < 2 * _NBLK))
    def _dot2():
        lhs = s1a_ref[...].reshape(T * B, K2)
        w2b = w2_ref[...].astype(jnp.bfloat16)
        a_ref[(s - _NBLK) % 2] = jnp.dot(
            lhs, w2b, preferred_element_type=jnp.float32).reshape(T, B, _NB)

    @pl.when(s >= _NBLK + 1)
    def _recur2():
        buf = (s - _NBLK - 1) % 2
        vc_ref[...] = jnp.zeros_like(vc_ref)
        m2_ref[...] = jnp.zeros_like(m2_ref)

        def lstep(t, c):
            v2 = vc_ref[...] * decay + a_ref[buf, t]
            mask = v2 > _THRESHOLD
            s2 = jnp.where(mask, 1.0, 0.0).astype(jnp.float32)
            vc_ref[...] = jnp.where(mask, 0.0, v2)
            m2_ref[...] += s2
            return c

        jax.lax.fori_loop(0, T, lstep, 0, unroll=3)
        m2_ref[...] = m2_ref[...] / jnp.float32(T)


def kernel(x, W1, W2):
    B = x.shape[0]
    T = _TIME_STEPS
    f32 = jnp.float32
    bf16 = jnp.bfloat16
    decay = jnp.exp(jnp.float32(-1.0 / _TAU_MEM)).reshape(1, 1)

    nb1 = _NBLK - 1

    mr, m1, m2 = pl.pallas_call(
        _body,
        grid=(2 * _NBLK + 1,),
        in_specs=[
            pl.BlockSpec((B, _INPUT_DIM), lambda s: (0, 0)),
            pl.BlockSpec((_INPUT_DIM, _NB),
                         lambda s: (0, jnp.minimum(s, nb1))),
            pl.BlockSpec((_HIDDEN_DIM, _NB),
                         lambda s: (0, jnp.clip(s - _NBLK, 0, nb1))),
            pl.BlockSpec((1, 1), lambda s: (0, 0)),
        ],
        out_specs=[
            pl.BlockSpec((B, _INPUT_DIM), lambda s: (0, 0)),
            pl.BlockSpec((B, _NB), lambda s: (0, jnp.clip(s - 1, 0, nb1))),
            pl.BlockSpec((B, _NB),
                         lambda s: (0, jnp.clip(s - _NBLK - 1, 0, nb1))),
        ],
        out_shape=[
            jax.ShapeDtypeStruct((B, _INPUT_DIM), f32),
            jax.ShapeDtypeStruct((B, _HIDDEN_DIM), f32),
            jax.ShapeDtypeStruct((B, _HIDDEN_DIM), f32),
        ],
        scratch_shapes=[
            pltpu.VMEM((T, B, _INPUT_DIM), bf16),
            pltpu.VMEM((T, B, _HIDDEN_DIM), bf16),
            pltpu.VMEM((2, T, B, _NB), f32),
            pltpu.VMEM((B, _INPUT_DIM), f32),
            pltpu.VMEM((B, _NB), f32),
        ],
        compiler_params=pltpu.CompilerParams(
            dimension_semantics=("arbitrary",),
            vmem_limit_bytes=64 * 1024 * 1024,
        ),
    )(x, W1, W2, decay)

    return (mr, m1, m2)


# speedup vs baseline: 1.0838x; 1.0028x over previous
import jax
import jax.numpy as jnp
from jax.experimental import pallas as pl
from jax.experimental.pallas import tpu as pltpu

_INPUT_DIM = 3072
_HIDDEN_DIM = 4096
_TIME_STEPS = 30
_TAU_MEM = 100.0
_THRESHOLD = 0.5
_INPUT_SCALE = 16.0
_NBLK = 16
_NB = _HIDDEN_DIM // _NBLK


def _body(x_ref, w1_ref, w2_ref, d_ref, mr_ref, m1_ref, m2_ref,
          sra_ref, s1a_ref, a_ref, vr_ref, vc_ref):
    s = pl.program_id(0)
    decay = d_ref[0, 0]
    T = _TIME_STEPS
    B = x_ref.shape[0]
    K1 = x_ref.shape[1]
    K2 = s1a_ref.shape[2]

    @pl.when(s == 0)
    def _retina():
        xr = x_ref[...]
        xmax = jnp.max(xr)
        x = jnp.where(xmax > 0, xr / xmax, xr) * _INPUT_SCALE
        vr_ref[...] = jnp.zeros_like(vr_ref)
        mr_ref[...] = jnp.zeros_like(mr_ref)

        def rstep(t, c):
            vr = vr_ref[...] * decay + x
            mask = vr > _THRESHOLD
            sr = jnp.where(mask, 1.0, 0.0).astype(jnp.float32)
            vr_ref[...] = jnp.where(mask, 0.0, vr)
            mr_ref[...] += sr
            sra_ref[t] = sr.astype(jnp.bfloat16)
            return c

        jax.lax.fori_loop(0, T, rstep, 0, unroll=5)
        mr_ref[...] = mr_ref[...] / jnp.float32(T)

    @pl.when(s < _NBLK)
    def _dot1():
        lhs = sra_ref[...].reshape(T * B, K1)
        w1b = w1_ref[...].astype(jnp.bfloat16)
        a_ref[s % 2] = jnp.dot(
            lhs, w1b, preferred_element_type=jnp.float32).reshape(T, B, _NB)

    @pl.when((s >= 1) & (s <= _NBLK))
    def _recur1():
        col = (s - 1) * _NB
        buf = (s - 1) % 2
        vc_ref[...] = jnp.zeros_like(vc_ref)
        m1_ref[...] = jnp.zeros_like(m1_ref)

        def lstep(t, c):
            v1 = vc_ref[...] * decay + a_ref[buf, t]
            mask = v1 > _THRESHOLD
            s1 = jnp.where(mask, 1.0, 0.0).astype(jnp.float32)
            vc_ref[...] = jnp.where(mask, 0.0, v1)
            m1_ref[...] += s1
            s1a_ref[t, :, pl.ds(col, _NB)] = s1.astype(jnp.bfloat16)
            return c

        jax.lax.fori_loop(0, T, lstep, 0, unroll=6)
        m1_ref[...] = m1_ref[...] / jnp.float32(T)

    @pl.when((s >= _NBLK) & (s < 2 * _NBLK))
    def _dot2():
        lhs = s1a_ref[...].reshape(T * B, K2)
        w2b = w2_ref[...].astype(jnp.bfloat16)
        a_ref[(s - _NBLK) % 2] = jnp.dot(
            lhs, w2b, preferred_element_type=jnp.float32).reshape(T, B, _NB)

    @pl.when(s >= _NBLK + 1)
    def _recur2():
        buf = (s - _NBLK - 1) % 2
        vc_ref[...] = jnp.zeros_like(vc_ref)
        m2_ref[...] = jnp.zeros_like(m2_ref)

        def lstep(t, c):
            v2 = vc_ref[...] * decay + a_ref[buf, t]
            mask = v2 > _THRESHOLD
            s2 = jnp.where(mask, 1.0, 0.0).astype(jnp.float32)
            vc_ref[...] = jnp.where(mask, 0.0, v2)
            m2_ref[...] += s2
            return c

        jax.lax.fori_loop(0, T, lstep, 0, unroll=6)
        m2_ref[...] = m2_ref[...] / jnp.float32(T)


def kernel(x, W1, W2):
    B = x.shape[0]
    T = _TIME_STEPS
    f32 = jnp.float32
    bf16 = jnp.bfloat16
    decay = jnp.exp(jnp.float32(-1.0 / _TAU_MEM)).reshape(1, 1)

    nb1 = _NBLK - 1

    mr, m1, m2 = pl.pallas_call(
        _body,
        grid=(2 * _NBLK + 1,),
        in_specs=[
            pl.BlockSpec((B, _INPUT_DIM), lambda s: (0, 0)),
            pl.BlockSpec((_INPUT_DIM, _NB),
                         lambda s: (0, jnp.minimum(s, nb1))),
            pl.BlockSpec((_HIDDEN_DIM, _NB),
                         lambda s: (0, jnp.clip(s - _NBLK, 0, nb1))),
            pl.BlockSpec((1, 1), lambda s: (0, 0)),
        ],
        out_specs=[
            pl.BlockSpec((B, _INPUT_DIM), lambda s: (0, 0)),
            pl.BlockSpec((B, _NB), lambda s: (0, jnp.clip(s - 1, 0, nb1))),
            pl.BlockSpec((B, _NB),
                         lambda s: (0, jnp.clip(s - _NBLK - 1, 0, nb1))),
        ],
        out_shape=[
            jax.ShapeDtypeStruct((B, _INPUT_DIM), f32),
            jax.ShapeDtypeStruct((B, _HIDDEN_DIM), f32),
            jax.ShapeDtypeStruct((B, _HIDDEN_DIM), f32),
        ],
        scratch_shapes=[
            pltpu.VMEM((T, B, _INPUT_DIM), bf16),
            pltpu.VMEM((T, B, _HIDDEN_DIM), bf16),
            pltpu.VMEM((2, T, B, _NB), f32),
            pltpu.VMEM((B, _INPUT_DIM), f32),
            pltpu.VMEM((B, _NB), f32),
        ],
        compiler_params=pltpu.CompilerParams(
            dimension_semantics=("arbitrary",),
            vmem_limit_bytes=64 * 1024 * 1024,
        ),
    )(x, W1, W2, decay)

    return (mr, m1, m2)


# speedup vs baseline: 1.0959x; 1.0112x over previous
import jax
import jax.numpy as jnp
from jax.experimental import pallas as pl
from jax.experimental.pallas import tpu as pltpu

_INPUT_DIM = 3072
_HIDDEN_DIM = 4096
_TIME_STEPS = 30
_TAU_MEM = 100.0
_THRESHOLD = 0.5
_INPUT_SCALE = 16.0
_NBLK = 16
_NB = _HIDDEN_DIM // _NBLK


def _body(x_ref, w1_ref, w2_ref, d_ref, mr_ref, m1_ref, m2_ref,
          sra_ref, s1a_ref, a_ref, vr_ref, vc_ref, mrb_ref, mb_ref):
    s = pl.program_id(0)
    decay = d_ref[0, 0]
    T = _TIME_STEPS
    B = x_ref.shape[0]
    K1 = x_ref.shape[1]
    K2 = s1a_ref.shape[2]

    @pl.when(s == 0)
    def _retina():
        xr = x_ref[...]
        xmax = jnp.max(xr)
        x = jnp.where(xmax > 0, xr / xmax, xr) * _INPUT_SCALE
        vr_ref[...] = jnp.zeros_like(vr_ref)
        mrb_ref[...] = jnp.zeros_like(mrb_ref)

        def rstep(t, c):
            vr = vr_ref[...] * decay + x
            mask = vr > _THRESHOLD
            sr = jnp.where(mask, 1.0, 0.0).astype(jnp.bfloat16)
            vr_ref[...] = jnp.where(mask, 0.0, vr)
            mrb_ref[...] += sr
            sra_ref[t] = sr
            return c

        jax.lax.fori_loop(0, T, rstep, 0, unroll=5)
        mr_ref[...] = mrb_ref[...].astype(jnp.float32) / jnp.float32(T)

    @pl.when(s < _NBLK)
    def _dot1():
        lhs = sra_ref[...].reshape(T * B, K1)
        w1b = w1_ref[...].astype(jnp.bfloat16)
        a_ref[s % 2] = jnp.dot(
            lhs, w1b, preferred_element_type=jnp.float32).reshape(T, B, _NB)

    @pl.when((s >= 1) & (s <= _NBLK))
    def _recur1():
        col = (s - 1) * _NB
        buf = (s - 1) % 2
        vc_ref[...] = jnp.zeros_like(vc_ref)
        mb_ref[...] = jnp.zeros_like(mb_ref)

        def lstep(t, c):
            v1 = vc_ref[...] * decay + a_ref[buf, t]
            mask = v1 > _THRESHOLD
            s1 = jnp.where(mask, 1.0, 0.0).astype(jnp.bfloat16)
            vc_ref[...] = jnp.where(mask, 0.0, v1)
            mb_ref[...] += s1
            s1a_ref[t, :, pl.ds(col, _NB)] = s1
            return c

        jax.lax.fori_loop(0, T, lstep, 0, unroll=6)
        m1_ref[...] = mb_ref[...].astype(jnp.float32) / jnp.float32(T)

    @pl.when((s >= _NBLK) & (s < 2 * _NBLK))
    def _dot2():
        lhs = s1a_ref[...].reshape(T * B, K2)
        w2b = w2_ref[...].astype(jnp.bfloat16)
        a_ref[(s - _NBLK) % 2] = jnp.dot(
            lhs, w2b, preferred_element_type=jnp.float32).reshape(T, B, _NB)

    @pl.when(s >= _NBLK + 1)
    def _recur2():
        buf = (s - _NBLK - 1) % 2
        vc_ref[...] = jnp.zeros_like(vc_ref)
        mb_ref[...] = jnp.zeros_like(mb_ref)

        def lstep(t, c):
            v2 = vc_ref[...] * decay + a_ref[buf, t]
            mask = v2 > _THRESHOLD
            s2 = jnp.where(mask, 1.0, 0.0).astype(jnp.bfloat16)
            vc_ref[...] = jnp.where(mask, 0.0, v2)
            mb_ref[...] += s2
            return c

        jax.lax.fori_loop(0, T, lstep, 0, unroll=6)
        m2_ref[...] = mb_ref[...].astype(jnp.float32) / jnp.float32(T)


def kernel(x, W1, W2):
    B = x.shape[0]
    T = _TIME_STEPS
    f32 = jnp.float32
    bf16 = jnp.bfloat16
    decay = jnp.exp(jnp.float32(-1.0 / _TAU_MEM)).reshape(1, 1)

    nb1 = _NBLK - 1

    mr, m1, m2 = pl.pallas_call(
        _body,
        grid=(2 * _NBLK + 1,),
        in_specs=[
            pl.BlockSpec((B, _INPUT_DIM), lambda s: (0, 0)),
            pl.BlockSpec((_INPUT_DIM, _NB),
                         lambda s: (0, jnp.minimum(s, nb1))),
            pl.BlockSpec((_HIDDEN_DIM, _NB),
                         lambda s: (0, jnp.clip(s - _NBLK, 0, nb1))),
            pl.BlockSpec((1, 1), lambda s: (0, 0)),
        ],
        out_specs=[
            pl.BlockSpec((B, _INPUT_DIM), lambda s: (0, 0)),
            pl.BlockSpec((B, _NB), lambda s: (0, jnp.clip(s - 1, 0, nb1))),
            pl.BlockSpec((B, _NB),
                         lambda s: (0, jnp.clip(s - _NBLK - 1, 0, nb1))),
        ],
        out_shape=[
            jax.ShapeDtypeStruct((B, _INPUT_DIM), f32),
            jax.ShapeDtypeStruct((B, _HIDDEN_DIM), f32),
            jax.ShapeDtypeStruct((B, _HIDDEN_DIM), f32),
        ],
        scratch_shapes=[
            pltpu.VMEM((T, B, _INPUT_DIM), bf16),
            pltpu.VMEM((T, B, _HIDDEN_DIM), bf16),
            pltpu.VMEM((2, T, B, _NB), f32),
            pltpu.VMEM((B, _INPUT_DIM), f32),
            pltpu.VMEM((B, _NB), f32),
            pltpu.VMEM((B, _INPUT_DIM), bf16),
            pltpu.VMEM((B, _NB), bf16),
        ],
        compiler_params=pltpu.CompilerParams(
            dimension_semantics=("arbitrary",),
            vmem_limit_bytes=64 * 1024 * 1024,
        ),
    )(x, W1, W2, decay)

    return (mr, m1, m2)


# speedup vs baseline: 1.0989x; 1.0027x over previous
import jax
import jax.numpy as jnp
from jax.experimental import pallas as pl
from jax.experimental.pallas import tpu as pltpu

_INPUT_DIM = 3072
_HIDDEN_DIM = 4096
_TIME_STEPS = 30
_TAU_MEM = 100.0
_THRESHOLD = 0.5
_INPUT_SCALE = 16.0
_NBLK = 16
_NB = _HIDDEN_DIM // _NBLK


def _body(x_ref, w1_ref, w2_ref, d_ref, mr_ref, m1_ref, m2_ref,
          sra_ref, s1a_ref, a_ref, vr_ref, vc_ref, mrb_ref, mb_ref):
    s = pl.program_id(0)
    decay = d_ref[0, 0]
    T = _TIME_STEPS
    B = x_ref.shape[0]
    K1 = x_ref.shape[1]
    K2 = s1a_ref.shape[2]

    @pl.when(s == 0)
    def _retina():
        xr = x_ref[...]
        xmax = jnp.max(xr)
        x = jnp.where(xmax > 0, xr / xmax, xr) * _INPUT_SCALE
        vr_ref[...] = jnp.zeros_like(vr_ref)
        mrb_ref[...] = jnp.zeros_like(mrb_ref)

        def rstep(t, c):
            vr = vr_ref[...] * decay + x
            mask = vr > _THRESHOLD
            sr = jnp.where(mask, 1.0, 0.0).astype(jnp.bfloat16)
            vr_ref[...] = jnp.where(mask, 0.0, vr)
            mrb_ref[...] += sr
            sra_ref[t] = sr
            return c

        jax.lax.fori_loop(0, T, rstep, 0, unroll=6)
        mr_ref[...] = mrb_ref[...].astype(jnp.float32) / jnp.float32(T)

    @pl.when(s < _NBLK)
    def _dot1():
        lhs = sra_ref[...].reshape(T * B, K1)
        w1b = w1_ref[...].astype(jnp.bfloat16)
        a_ref[s % 2] = jnp.dot(
            lhs, w1b, preferred_element_type=jnp.float32).reshape(T, B, _NB)

    @pl.when((s >= 1) & (s <= _NBLK))
    def _recur1():
        col = (s - 1) * _NB
        buf = (s - 1) % 2
        vc_ref[...] = jnp.zeros_like(vc_ref)
        mb_ref[...] = jnp.zeros_like(mb_ref)

        def lstep(t, c):
            v1 = vc_ref[...] * decay + a_ref[buf, t]
            mask = v1 > _THRESHOLD
            s1 = jnp.where(mask, 1.0, 0.0).astype(jnp.bfloat16)
            vc_ref[...] = jnp.where(mask, 0.0, v1)
            mb_ref[...] += s1
            s1a_ref[t, :, pl.ds(col, _NB)] = s1
            return c

        jax.lax.fori_loop(0, T, lstep, 0, unroll=10)
        m1_ref[...] = mb_ref[...].astype(jnp.float32) / jnp.float32(T)

    @pl.when((s >= _NBLK) & (s < 2 * _NBLK))
    def _dot2():
        lhs = s1a_ref[...].reshape(T * B, K2)
        w2b = w2_ref[...].astype(jnp.bfloat16)
        a_ref[(s - _NBLK) % 2] = jnp.dot(
            lhs, w2b, preferred_element_type=jnp.float32).reshape(T, B, _NB)

    @pl.when(s >= _NBLK + 1)
    def _recur2():
        buf = (s - _NBLK - 1) % 2
        vc_ref[...] = jnp.zeros_like(vc_ref)
        mb_ref[...] = jnp.zeros_like(mb_ref)

        def lstep(t, c):
            v2 = vc_ref[...] * decay + a_ref[buf, t]
            mask = v2 > _THRESHOLD
            s2 = jnp.where(mask, 1.0, 0.0).astype(jnp.bfloat16)
            vc_ref[...] = jnp.where(mask, 0.0, v2)
            mb_ref[...] += s2
            return c

        jax.lax.fori_loop(0, T, lstep, 0, unroll=10)
        m2_ref[...] = mb_ref[...].astype(jnp.float32) / jnp.float32(T)


def kernel(x, W1, W2):
    B = x.shape[0]
    T = _TIME_STEPS
    f32 = jnp.float32
    bf16 = jnp.bfloat16
    decay = jnp.exp(jnp.float32(-1.0 / _TAU_MEM)).reshape(1, 1)

    nb1 = _NBLK - 1

    mr, m1, m2 = pl.pallas_call(
        _body,
        grid=(2 * _NBLK + 1,),
        in_specs=[
            pl.BlockSpec((B, _INPUT_DIM), lambda s: (0, 0)),
            pl.BlockSpec((_INPUT_DIM, _NB),
                         lambda s: (0, jnp.minimum(s, nb1))),
            pl.BlockSpec((_HIDDEN_DIM, _NB),
                         lambda s: (0, jnp.clip(s - _NBLK, 0, nb1))),
            pl.BlockSpec((1, 1), lambda s: (0, 0)),
        ],
        out_specs=[
            pl.BlockSpec((B, _INPUT_DIM), lambda s: (0, 0)),
            pl.BlockSpec((B, _NB), lambda s: (0, jnp.clip(s - 1, 0, nb1))),
            pl.BlockSpec((B, _NB),
                         lambda s: (0, jnp.clip(s - _NBLK - 1, 0, nb1))),
        ],
        out_shape=[
            jax.ShapeDtypeStruct((B, _INPUT_DIM), f32),
            jax.ShapeDtypeStruct((B, _HIDDEN_DIM), f32),
            jax.ShapeDtypeStruct((B, _HIDDEN_DIM), f32),
        ],
        scratch_shapes=[
            pltpu.VMEM((T, B, _INPUT_DIM), bf16),
            pltpu.VMEM((T, B, _HIDDEN_DIM), bf16),
            pltpu.VMEM((2, T, B, _NB), f32),
            pltpu.VMEM((B, _INPUT_DIM), f32),
            pltpu.VMEM((B, _NB), f32),
            pltpu.VMEM((B, _INPUT_DIM), bf16),
            pltpu.VMEM((B, _NB), bf16),
        ],
        compiler_params=pltpu.CompilerParams(
            dimension_semantics=("arbitrary",),
            vmem_limit_bytes=64 * 1024 * 1024,
        ),
    )(x, W1, W2, decay)

    return (mr, m1, m2)
